# Initial kernel scaffold; baseline (speedup 1.0000x reference)
import functools

import jax
import jax.numpy as jnp
from jax import lax
from jax.experimental import pallas as pl
from jax.experimental.pallas import tpu as pltpu
from jax.experimental.pallas import tpu_sc as plsc

N = 884
FD = 128
E = 56576
NN = N * N
PNN = 786432
CH = 2 * PNN // 16
ZB = 2048
NW = 32
EPW = 5376
NCH = EPW // 128
TOT = NW * EPW
HALF = 16 * EPW


def _sc_body(dcat_hbm, gidx_hbm, sidx_hbm, out_hbm, g_v, s_v, vals_v, zbuf, a_sh):
    c = lax.axis_index("c")
    s = lax.axis_index("s")
    w = c * 16 + s

    def _zb(i, carry):
        zbuf[pl.ds(i * 16, 16)] = jnp.zeros((16,), jnp.float32)
        return carry

    lax.fori_loop(0, ZB // 16, _zb, 0)

    base = pl.multiple_of(s * CH, ZB)

    def _zs(k, carry):
        pltpu.sync_copy(zbuf, a_sh.at[pl.ds(base + k * ZB, ZB)])
        return carry

    lax.fori_loop(0, CH // ZB, _zs, 0)

    pltpu.sync_copy(gidx_hbm.at[w], g_v)
    pltpu.sync_copy(sidx_hbm.at[w], s_v)
    plsc.subcore_barrier()

    def _chunk(j, carry):
        pltpu.sync_copy(dcat_hbm.at[g_v.at[j]], vals_v.at[j])
        pltpu.sync_copy(vals_v.at[j], a_sh.at[s_v.at[j]], add=True)
        return carry

    lax.fori_loop(0, NCH, _chunk, 0)
    plsc.subcore_barrier()

    pltpu.sync_copy(a_sh.at[pl.ds(base, CH)], out_hbm.at[c, pl.ds(base, CH)])


_sc_build_adj = functools.partial(
    pl.kernel,
    out_type=jax.ShapeDtypeStruct((2, 2 * PNN), jnp.float32),
    mesh=plsc.VectorSubcoreMesh(core_axis_name="c", subcore_axis_name="s"),
    scratch_types=[
        pltpu.VMEM((NCH, 128), jnp.int32),
        pltpu.VMEM((NCH, 128), jnp.int32),
        pltpu.VMEM((NCH, 128), jnp.float32),
        pltpu.VMEM((ZB,), jnp.float32),
        pltpu.VMEM_SHARED((2 * PNN,), jnp.float32),
    ],
)(_sc_body)


def _tc_body(x_ref, a0_ref, a1a_ref, a1b_ref, a2_ref,
             w1_refs, b1_refs, w2_refs, b2_refs,
             f1w_ref, f1b_ref, f2w_ref, f2b_ref, cw_ref, cb_ref, out_ref):
    x = x_ref[...]
    rowi = lax.broadcasted_iota(jnp.int32, (N, N), 0)
    coli = lax.broadcasted_iota(jnp.int32, (N, N), 1)
    eye = jnp.where(rowi == coli, jnp.float32(1.0), jnp.float32(0.0))

    ys = []
    msums = []
    for v in range(3):
        if v == 0:
            A = a0_ref[...]
        elif v == 1:
            A = a1a_ref[...] + a1b_ref[...]
        else:
            A = a2_ref[...]
        deg = jnp.sum(A, axis=1, keepdims=True) + 1.0
        dinv = jnp.where(deg > 0, lax.rsqrt(deg), 0.0)
        Ai = A + eye
        h = x
        for W_ref, b_ref in ((w1_refs[v], b1_refs[v]), (w2_refs[v], b2_refs[v])):
            hw = jnp.dot(h, W_ref[...], preferred_element_type=jnp.float32)
            agg = dinv * jnp.dot(Ai, dinv * hw,
                                 preferred_element_type=jnp.float32)
            h = jnp.maximum(agg + b_ref[...], 0.0)
            ys.append(h)
            msums.append(jnp.sum(h))

    m = jnp.concatenate([t.reshape(1, 1) for t in msums], axis=1) / (N * FD)
    ca = jnp.maximum(
        jnp.dot(m, f1w_ref[...], preferred_element_type=jnp.float32)
        + f1b_ref[...], 0.0)
    ca = jax.nn.sigmoid(
        jnp.dot(ca, f2w_ref[...], preferred_element_type=jnp.float32)
        + f2b_ref[...])

    acc = jnp.full((N, FD), cb_ref[0, 0], jnp.float32)
    for j in range(6):
        acc = acc + cw_ref[0, j] * jnp.maximum(ca[0, j] * ys[j], 0.0)
    out_ref[...] = acc


def kernel(x_d, di_gua, di_cos, di_sem, W_t1, b_t1, W_t2, b_t2, W_s1, b_s1,
           W_s2, b_s2, W_g1, b_g1, W_g2, b_g2, fc1_W, fc1_b, fc2_W, fc2_b,
           cnn_W, cnn_b, di_gua_edges, di_cos_edges, di_sem_edges):
    f32 = jnp.float32

    srcs = jnp.concatenate([di_gua_edges[0], di_cos_edges[0], di_sem_edges[0]])
    dsts = jnp.concatenate([di_gua_edges[1], di_cos_edges[1], di_sem_edges[1]])
    p = jnp.arange(3 * E, dtype=jnp.int32)
    voff = (p // E) * NN
    g = srcs * N + dsts + voff
    slot = jnp.where(p < E, 0, jnp.where(p < HALF, PNN,
                     jnp.where(p < 2 * E, 0, PNN))).astype(jnp.int32)
    sidx = dsts * N + srcs + slot
    npad = TOT - 3 * E
    g_pad = jnp.concatenate([g, jnp.zeros((npad,), jnp.int32)])
    s_pad = jnp.concatenate([sidx, jnp.full((npad,), NN, jnp.int32)])
    gidx = g_pad.reshape(NW, NCH, 128)
    sidx3 = s_pad.reshape(NW, NCH, 128)
    dcat = jnp.concatenate(
        [di_gua.reshape(-1), di_cos.reshape(-1), di_sem.reshape(-1)])

    sc_out = _sc_build_adj(dcat, gidx, sidx3)

    a0 = sc_out[0, :NN].reshape(N, N)
    a1a = sc_out[0, PNN:PNN + NN].reshape(N, N)
    a1b = sc_out[1, :NN].reshape(N, N)
    a2 = sc_out[1, PNN:PNN + NN].reshape(N, N)

    out = pl.pallas_call(
        _tc_body,
        out_shape=jax.ShapeDtypeStruct((N, FD), f32),
    )(
        x_d, a0, a1a, a1b, a2,
        [W_t1, W_s1, W_g1], [b_t1.reshape(1, FD), b_s1.reshape(1, FD),
                             b_g1.reshape(1, FD)],
        [W_t2, W_s2, W_g2], [b_t2.reshape(1, FD), b_s2.reshape(1, FD),
                             b_g2.reshape(1, FD)],
        fc1_W, fc1_b.reshape(1, 30), fc2_W, fc2_b.reshape(1, 6),
        cnn_W.reshape(1, 6), cnn_b.reshape(1, 1),
    )
    return out


# speedup vs baseline: 16.3745x; 16.3745x over previous
import functools

import jax
import jax.numpy as jnp
from jax import lax
from jax.experimental import pallas as pl
from jax.experimental.pallas import tpu as pltpu
from jax.experimental.pallas import tpu_sc as plsc

N = 884
FD = 128
E = 56576
NN = N * N
PNN = 786432
CH = 2 * PNN // 16
ZB = 2048
NW = 32
EPW = 5376
NCH = EPW // 128
TOT = NW * EPW
HALF = 16 * EPW


def _sc_body(dcat_hbm, gidx_hbm, sidx_hbm, out_hbm, g_v, s_v, vals_v, zbuf, a_sh):
    c = lax.axis_index("c")
    s = lax.axis_index("s")
    w = c * 16 + s

    def _zb(i, carry):
        zbuf[pl.ds(i * 16, 16)] = jnp.zeros((16,), jnp.float32)
        return carry

    lax.fori_loop(0, ZB // 16, _zb, 0)

    base = pl.multiple_of(s * CH, ZB)

    def _zs(k, carry):
        pltpu.sync_copy(zbuf, a_sh.at[pl.ds(base + k * ZB, ZB)])
        return carry

    lax.fori_loop(0, CH // ZB, _zs, 0)

    pltpu.sync_copy(gidx_hbm.at[w], g_v)
    pltpu.sync_copy(sidx_hbm.at[w], s_v)
    plsc.subcore_barrier()

    def _chunk(j, carry):
        pltpu.sync_copy(dcat_hbm.at[g_v.at[j]], vals_v.at[j])
        pltpu.sync_copy(vals_v.at[j], a_sh.at[s_v.at[j]], add=True)
        return carry

    lax.fori_loop(0, NCH, _chunk, 0)
    plsc.subcore_barrier()

    pltpu.sync_copy(a_sh.at[pl.ds(base, CH)], out_hbm.at[c, pl.ds(base, CH)])


@functools.cache
def _sc_build_adj():
    return pl.kernel(
        _sc_body,
        out_type=jax.ShapeDtypeStruct((2, 2 * PNN), jnp.float32),
        mesh=plsc.VectorSubcoreMesh(core_axis_name="c", subcore_axis_name="s"),
        scratch_types=[
            pltpu.VMEM((NCH, 128), jnp.int32),
            pltpu.VMEM((NCH, 128), jnp.int32),
            pltpu.VMEM((NCH, 128), jnp.float32),
            pltpu.VMEM((ZB,), jnp.float32),
            pltpu.VMEM_SHARED((2 * PNN,), jnp.float32),
        ],
    )


def _tc_body(x_ref, a0_ref, a1a_ref, a1b_ref, a2_ref,
             w1_refs, b1_refs, w2_refs, b2_refs,
             f1w_ref, f1b_ref, f2w_ref, f2b_ref, cw_ref, cb_ref, out_ref):
    x = x_ref[...]
    rowi = lax.broadcasted_iota(jnp.int32, (N, N), 0)
    coli = lax.broadcasted_iota(jnp.int32, (N, N), 1)
    eye = jnp.where(rowi == coli, jnp.float32(1.0), jnp.float32(0.0))

    ys = []
    msums = []
    for v in range(3):
        if v == 0:
            A = a0_ref[...]
        elif v == 1:
            A = a1a_ref[...] + a1b_ref[...]
        else:
            A = a2_ref[...]
        deg = jnp.sum(A, axis=1, keepdims=True) + 1.0
        dinv = jnp.where(deg > 0, lax.rsqrt(deg), 0.0)
        Ai = A + eye
        h = x
        for W_ref, b_ref in ((w1_refs[v], b1_refs[v]), (w2_refs[v], b2_refs[v])):
            hw = jnp.dot(h, W_ref[...], preferred_element_type=jnp.float32)
            agg = dinv * jnp.dot(Ai, dinv * hw,
                                 preferred_element_type=jnp.float32)
            h = jnp.maximum(agg + b_ref[...], 0.0)
            ys.append(h)
            msums.append(jnp.sum(h))

    m = jnp.concatenate([t.reshape(1, 1) for t in msums], axis=1) / (N * FD)
    ca = jnp.maximum(
        jnp.dot(m, f1w_ref[...], preferred_element_type=jnp.float32)
        + f1b_ref[...], 0.0)
    ca = jax.nn.sigmoid(
        jnp.dot(ca, f2w_ref[...], preferred_element_type=jnp.float32)
        + f2b_ref[...])

    acc = jnp.full((N, FD), cb_ref[0, 0], jnp.float32)
    for j in range(6):
        acc = acc + cw_ref[0, j] * jnp.maximum(ca[0, j] * ys[j], 0.0)
    out_ref[...] = acc


def kernel(x_d, di_gua, di_cos, di_sem, W_t1, b_t1, W_t2, b_t2, W_s1, b_s1,
           W_s2, b_s2, W_g1, b_g1, W_g2, b_g2, fc1_W, fc1_b, fc2_W, fc2_b,
           cnn_W, cnn_b, di_gua_edges, di_cos_edges, di_sem_edges):
    f32 = jnp.float32

    srcs = jnp.concatenate([di_gua_edges[0], di_cos_edges[0], di_sem_edges[0]])
    dsts = jnp.concatenate([di_gua_edges[1], di_cos_edges[1], di_sem_edges[1]])
    p = jnp.arange(3 * E, dtype=jnp.int32)
    voff = (p // E) * NN
    g = srcs * N + dsts + voff
    slot = jnp.where(p < E, 0, jnp.where(p < HALF, PNN,
                     jnp.where(p < 2 * E, 0, PNN))).astype(jnp.int32)
    sidx = dsts * N + srcs + slot
    npad = TOT - 3 * E
    g_pad = jnp.concatenate([g, jnp.zeros((npad,), jnp.int32)])
    s_pad = jnp.concatenate([sidx, jnp.full((npad,), NN, jnp.int32)])
    gidx = g_pad.reshape(NW, NCH, 128)
    sidx3 = s_pad.reshape(NW, NCH, 128)
    dcat = jnp.concatenate(
        [di_gua.reshape(-1), di_cos.reshape(-1), di_sem.reshape(-1)])

    sc_out = _sc_build_adj()(dcat, gidx, sidx3)

    a0 = sc_out[0, :NN].reshape(N, N)
    a1a = sc_out[0, PNN:PNN + NN].reshape(N, N)
    a1b = sc_out[1, :NN].reshape(N, N)
    a2 = sc_out[1, PNN:PNN + NN].reshape(N, N)

    out = pl.pallas_call(
        _tc_body,
        out_shape=jax.ShapeDtypeStruct((N, FD), f32),
    )(
        x_d, a0, a1a, a1b, a2,
        [W_t1, W_s1, W_g1], [b_t1.reshape(1, FD), b_s1.reshape(1, FD),
                             b_g1.reshape(1, FD)],
        [W_t2, W_s2, W_g2], [b_t2.reshape(1, FD), b_s2.reshape(1, FD),
                             b_g2.reshape(1, FD)],
        fc1_W, fc1_b.reshape(1, 30), fc2_W, fc2_b.reshape(1, 6),
        cnn_W.reshape(1, 6), cnn_b.reshape(1, 1),
    )
    return out


# speedup vs baseline: 43.7443x; 2.6715x over previous
import functools

import jax
import jax.numpy as jnp
from jax import lax
from jax.experimental import pallas as pl
from jax.experimental.pallas import tpu as pltpu
from jax.experimental.pallas import tpu_sc as plsc

N = 884
FD = 128
E = 56576
NN = N * N
PNN = 786432
CH = 2 * PNN // 16
ZB = 8192
NZC = CH // ZB
NW = 32
EPW = 5376
NCH = EPW // 128
TOT = NW * EPW
HALF = 16 * EPW

DN0 = (((0,), (0,)), ((), ()))


def _sc_body(sidx_hbm, out_hbm, s_v, ones_v, zbuf, a_sh, zsem):
    c = lax.axis_index("c")
    s = lax.axis_index("s")
    w = c * 16 + s

    def _zb(i, carry):
        zbuf[pl.ds(i * 16, 16)] = jnp.zeros((16,), jnp.float32)
        return carry

    lax.fori_loop(0, ZB // 16, _zb, 0)
    for i in range(8):
        ones_v[pl.ds(i * 16, 16)] = jnp.ones((16,), jnp.float32)

    base = pl.multiple_of(s * CH, ZB)
    zcopies = [
        pltpu.async_copy(zbuf, a_sh.at[pl.ds(base + k * ZB, ZB)], zsem)
        for k in range(NZC)
    ]
    pltpu.sync_copy(sidx_hbm.at[w], s_v)
    for cp in zcopies:
        cp.wait()
    plsc.subcore_barrier()

    def _chunk(j, carry):
        pltpu.sync_copy(ones_v, a_sh.at[s_v.at[j]], add=True)
        return carry

    lax.fori_loop(0, NCH, _chunk, 0)
    plsc.subcore_barrier()

    pltpu.sync_copy(a_sh.at[pl.ds(base, CH)],
                    out_hbm.at[pl.ds(c * 2 * PNN + base, CH)])


@functools.cache
def _sc_build_counts():
    return pl.kernel(
        _sc_body,
        out_type=jax.ShapeDtypeStruct((4 * PNN,), jnp.float32),
        mesh=plsc.VectorSubcoreMesh(core_axis_name="c", subcore_axis_name="s"),
        scratch_types=[
            pltpu.VMEM((NCH, 128), jnp.int32),
            pltpu.VMEM((128,), jnp.float32),
            pltpu.VMEM((ZB,), jnp.float32),
            pltpu.VMEM_SHARED((2 * PNN,), jnp.float32),
            pltpu.SemaphoreType.DMA,
        ],
    )


def _tc_body(x_ref, c0_ref, c1a_ref, c1b_ref, c2_ref, d0_ref, d1_ref, d2_ref,
             w1_refs, b1_refs, w2_refs, b2_refs,
             f1w_ref, f1b_ref, f2w_ref, f2b_ref, cw_ref, cb_ref, out_ref):
    x = x_ref[...]
    ones_col = jnp.ones((N, 1), jnp.float32)

    ys = []
    msums = []
    for v in range(3):
        if v == 0:
            B = c0_ref[...] * d0_ref[...]
        elif v == 1:
            B = (c1a_ref[...] + c1b_ref[...]) * d1_ref[...]
        else:
            B = c2_ref[...] * d2_ref[...]
        deg = lax.dot_general(B, ones_col, DN0,
                              preferred_element_type=jnp.float32) + 1.0
        dinv = jnp.where(deg > 0, lax.rsqrt(deg), 0.0)
        h = x
        for W_ref, b_ref in ((w1_refs[v], b1_refs[v]), (w2_refs[v], b2_refs[v])):
            hw = jnp.dot(h, W_ref[...], preferred_element_type=jnp.float32)
            z = dinv * hw
            agg = lax.dot_general(B, z, DN0,
                                  preferred_element_type=jnp.float32) + z
            h = jnp.maximum(dinv * agg + b_ref[...], 0.0)
            ys.append(h)
            msums.append(jnp.sum(h))

    m = jnp.concatenate([t.reshape(1, 1) for t in msums], axis=1) / (N * FD)
    ca = jnp.maximum(
        jnp.dot(m, f1w_ref[...], preferred_element_type=jnp.float32)
        + f1b_ref[...], 0.0)
    ca = jax.nn.sigmoid(
        jnp.dot(ca, f2w_ref[...], preferred_element_type=jnp.float32)
        + f2b_ref[...])

    acc = jnp.full((N, FD), cb_ref[0, 0], jnp.float32)
    for j in range(6):
        acc = acc + cw_ref[0, j] * jnp.maximum(ca[0, j] * ys[j], 0.0)
    out_ref[...] = acc


def kernel(x_d, di_gua, di_cos, di_sem, W_t1, b_t1, W_t2, b_t2, W_s1, b_s1,
           W_s2, b_s2, W_g1, b_g1, W_g2, b_g2, fc1_W, fc1_b, fc2_W, fc2_b,
           cnn_W, cnn_b, di_gua_edges, di_cos_edges, di_sem_edges):
    f32 = jnp.float32

    srcs = jnp.concatenate([di_gua_edges[0], di_cos_edges[0], di_sem_edges[0]])
    dsts = jnp.concatenate([di_gua_edges[1], di_cos_edges[1], di_sem_edges[1]])
    p = jnp.arange(3 * E, dtype=jnp.int32)
    slot = jnp.where(p < E, 0, jnp.where(p < HALF, PNN,
                     jnp.where(p < 2 * E, 0, PNN))).astype(jnp.int32)
    sidx = srcs * N + dsts + slot
    npad = TOT - 3 * E
    s_pad = jnp.concatenate([sidx, jnp.full((npad,), NN, jnp.int32)])
    sidx3 = s_pad.reshape(NW, NCH, 128)

    sc_out = _sc_build_counts()(sidx3)

    c0 = sc_out[:NN].reshape(N, N)
    c1a = sc_out[PNN:PNN + NN].reshape(N, N)
    c1b = sc_out[2 * PNN:2 * PNN + NN].reshape(N, N)
    c2 = sc_out[3 * PNN:3 * PNN + NN].reshape(N, N)

    out = pl.pallas_call(
        _tc_body,
        out_shape=jax.ShapeDtypeStruct((N, FD), f32),
    )(
        x_d, c0, c1a, c1b, c2, di_gua, di_cos, di_sem,
        [W_t1, W_s1, W_g1], [b_t1.reshape(1, FD), b_s1.reshape(1, FD),
                             b_g1.reshape(1, FD)],
        [W_t2, W_s2, W_g2], [b_t2.reshape(1, FD), b_s2.reshape(1, FD),
                             b_g2.reshape(1, FD)],
        fc1_W, fc1_b.reshape(1, 30), fc2_W, fc2_b.reshape(1, 6),
        cnn_W.reshape(1, 6), cnn_b.reshape(1, 1),
    )
    return out


# speedup vs baseline: 66.1013x; 1.5111x over previous
import functools

import jax
import jax.numpy as jnp
from jax import lax
from jax.experimental import pallas as pl
from jax.experimental.pallas import tpu as pltpu
from jax.experimental.pallas import tpu_sc as plsc

N = 884
FD = 128
E = 56576
NP = 888
KB = 7
SLOT = KB * NP * 128
CH = 2 * SLOT // 16
ZB = 4144
NZC = CH // ZB
NW = 32
EPW = 5376
NCH = EPW // 128
TOT = NW * EPW
HALF = 16 * EPW
DUMP = N * 128


def _sc_body(sidx_hbm, out_hbm, s_v, ones_v, zbuf, a_sh, zsem):
    c = lax.axis_index("c")
    s = lax.axis_index("s")
    w = c * 16 + s

    def _zb(i, carry):
        zbuf[pl.ds(i * 16, 16)] = jnp.zeros((16,), jnp.float32)
        return carry

    lax.fori_loop(0, ZB // 16, _zb, 0)
    for i in range(8):
        ones_v[pl.ds(i * 16, 16)] = jnp.ones((16,), jnp.float32)

    base = pl.multiple_of(s * CH, 8)
    zcopies = [
        pltpu.async_copy(zbuf, a_sh.at[pl.ds(base + k * ZB, ZB)], zsem)
        for k in range(NZC)
    ]
    pltpu.sync_copy(sidx_hbm.at[w], s_v)
    for cp in zcopies:
        cp.wait()
    plsc.subcore_barrier()

    def _chunk(j, carry):
        pltpu.sync_copy(ones_v, a_sh.at[s_v.at[j]], add=True)
        return carry

    lax.fori_loop(0, NCH, _chunk, 0)
    plsc.subcore_barrier()

    pltpu.sync_copy(a_sh.at[pl.ds(base, CH)],
                    out_hbm.at[pl.ds(c * 2 * SLOT + base, CH)])


@functools.cache
def _sc_build_counts():
    return pl.kernel(
        _sc_body,
        out_type=jax.ShapeDtypeStruct((4 * SLOT,), jnp.float32),
        mesh=plsc.VectorSubcoreMesh(core_axis_name="c", subcore_axis_name="s"),
        scratch_types=[
            pltpu.VMEM((NCH, 128), jnp.int32),
            pltpu.VMEM((128,), jnp.float32),
            pltpu.VMEM((ZB,), jnp.float32),
            pltpu.VMEM_SHARED((2 * SLOT,), jnp.float32),
            pltpu.SemaphoreType.DMA,
        ],
    )


def _tc_body(x_ref, scr_ref, dk0_ref, dk1_ref, dk2_ref,
             w1_refs, b1_refs, w2_refs, b2_refs,
             f1w_ref, f1b_ref, f2w_ref, f2b_ref, cw_ref, cb_ref, out_ref):
    x = x_ref[...]
    rowmask = jnp.where(
        lax.broadcasted_iota(jnp.int32, (NP, 1), 0) < N, 1.0, 0.0)

    ys = []
    msums = []
    for v in range(3):
        if v == 0:
            C = scr_ref[0]
            Dk = dk0_ref[...]
        elif v == 1:
            C = scr_ref[1] + scr_ref[2]
            Dk = dk1_ref[...]
        else:
            C = scr_ref[3]
            Dk = dk2_ref[...]
        B = C * Dk
        deg = jnp.sum(jnp.sum(B, axis=2, keepdims=True), axis=0) + 1.0
        dinv = jnp.where(deg > 0, lax.rsqrt(deg), 0.0)
        h = x
        for W_ref, b_ref in ((w1_refs[v], b1_refs[v]), (w2_refs[v], b2_refs[v])):
            hw = jnp.dot(h, W_ref[...], preferred_element_type=jnp.float32)
            z = dinv * hw
            zp = jnp.concatenate(
                [z, jnp.zeros((KB * 128 - NP, FD), jnp.float32)], axis=0)
            agg = z
            for k in range(KB):
                agg = agg + jnp.dot(B[k], zp[128 * k:128 * (k + 1), :],
                                    preferred_element_type=jnp.float32)
            h = jnp.maximum(dinv * agg + b_ref[...], 0.0) * rowmask
            ys.append(h)
            msums.append(jnp.sum(h))

    m = jnp.concatenate([t.reshape(1, 1) for t in msums], axis=1) / (N * FD)
    ca = jnp.maximum(
        jnp.dot(m, f1w_ref[...], preferred_element_type=jnp.float32)
        + f1b_ref[...], 0.0)
    ca = jax.nn.sigmoid(
        jnp.dot(ca, f2w_ref[...], preferred_element_type=jnp.float32)
        + f2b_ref[...])

    acc = jnp.full((NP, FD), cb_ref[0, 0], jnp.float32)
    for j in range(6):
        acc = acc + cw_ref[0, j] * jnp.maximum(ca[0, j] * ys[j], 0.0)
    out_ref[...] = acc[:N, :]


def _blocked(D):
    dt = jnp.pad(D.T, ((0, NP - N), (0, KB * 128 - N)))
    return jnp.transpose(dt.reshape(NP, KB, 128), (1, 0, 2))


def kernel(x_d, di_gua, di_cos, di_sem, W_t1, b_t1, W_t2, b_t2, W_s1, b_s1,
           W_s2, b_s2, W_g1, b_g1, W_g2, b_g2, fc1_W, fc1_b, fc2_W, fc2_b,
           cnn_W, cnn_b, di_gua_edges, di_cos_edges, di_sem_edges):
    f32 = jnp.float32

    srcs = jnp.concatenate([di_gua_edges[0], di_cos_edges[0], di_sem_edges[0]])
    dsts = jnp.concatenate([di_gua_edges[1], di_cos_edges[1], di_sem_edges[1]])
    p = jnp.arange(3 * E, dtype=jnp.int32)
    slot = jnp.where(p < E, 0, jnp.where(p < HALF, SLOT,
                     jnp.where(p < 2 * E, 0, SLOT))).astype(jnp.int32)
    sidx = (srcs // 128) * (NP * 128) + dsts * 128 + (srcs % 128) + slot
    npad = TOT - 3 * E
    s_pad = jnp.concatenate([sidx, jnp.full((npad,), DUMP, jnp.int32)])
    sidx3 = s_pad.reshape(NW, NCH, 128)

    dk0, dk1, dk2 = _blocked(di_gua), _blocked(di_cos), _blocked(di_sem)
    xp = jnp.pad(x_d, ((0, NP - N), (0, 0)))

    sc_out = _sc_build_counts()(sidx3)
    scr = sc_out.reshape(4, KB, NP, 128)

    out = pl.pallas_call(
        _tc_body,
        out_shape=jax.ShapeDtypeStruct((N, FD), f32),
    )(
        xp, scr, dk0, dk1, dk2,
        [W_t1, W_s1, W_g1], [b_t1.reshape(1, FD), b_s1.reshape(1, FD),
                             b_g1.reshape(1, FD)],
        [W_t2, W_s2, W_g2], [b_t2.reshape(1, FD), b_s2.reshape(1, FD),
                             b_g2.reshape(1, FD)],
        fc1_W, fc1_b.reshape(1, 30), fc2_W, fc2_b.reshape(1, 6),
        cnn_W.reshape(1, 6), cnn_b.reshape(1, 1),
    )
    return out
